# baseline (speedup 1.0000x reference)
import jax
import jax.numpy as jnp
from jax.experimental import pallas as pl
from jax.experimental.pallas import tpu as pltpu

B = 1024
D = 64
K = 100000
BLK_K = 6144
NT = pl.cdiv(K, BLK_K)


def _logits_kernel(xt_ref, pt_ref, o_ref):
    o_ref[...] = jax.lax.dot_general(
        pt_ref[...],
        xt_ref[...],
        dimension_numbers=(((0,), (0,)), ((), ())),
        preferred_element_type=jnp.float32,
    )


def kernel(x, proto):
    out_t = pl.pallas_call(
        _logits_kernel,
        grid=(NT,),
        in_specs=[
            pl.BlockSpec((D, B), lambda k: (0, 0)),
            pl.BlockSpec((D, BLK_K), lambda k: (0, k)),
        ],
        out_specs=pl.BlockSpec((BLK_K, B), lambda k: (k, 0)),
        out_shape=jax.ShapeDtypeStruct((K, B), jnp.float32),
        compiler_params=pltpu.CompilerParams(
            dimension_semantics=("arbitrary",),
        ),
    )(x.T, proto.T)
    return out_t.T


# speedup vs baseline: 1.0221x; 1.0221x over previous
import jax
import jax.numpy as jnp
from jax.experimental import pallas as pl
from jax.experimental.pallas import tpu as pltpu

B = 1024
D = 64
K = 100000
BLK_K = 4096
NT = pl.cdiv(K, BLK_K)


def _logits_kernel(xt_ref, pt_ref, o_ref):
    o_ref[...] = jax.lax.dot_general(
        pt_ref[...],
        xt_ref[...],
        dimension_numbers=(((0,), (0,)), ((), ())),
        preferred_element_type=jnp.float32,
    )


def kernel(x, proto):
    out_t = pl.pallas_call(
        _logits_kernel,
        grid=(NT,),
        in_specs=[
            pl.BlockSpec((D, B), lambda k: (0, 0)),
            pl.BlockSpec((D, BLK_K), lambda k: (0, k)),
        ],
        out_specs=pl.BlockSpec((BLK_K, B), lambda k: (k, 0)),
        out_shape=jax.ShapeDtypeStruct((K, B), jnp.float32),
        compiler_params=pltpu.CompilerParams(
            dimension_semantics=("arbitrary",),
        ),
    )(x.T, proto.T)
    return out_t.T
